# baseline (speedup 1.0000x reference)
import functools

import jax
import jax.numpy as jnp
from jax.experimental import pallas as pl
from jax.experimental.pallas import tpu as pltpu


def _fused_body(x_ref, w1_ref, b1_ref, w2_ref, b2_ref, cb_ref,
                cw_ref, vq_ref, h_ref):
    k = pl.program_id(0)

    @pl.when(k == 0)
    def _compute_h():
        h = jax.lax.dot_general(
            x_ref[...], w1_ref[...],
            (((1,), (1,)), ((), ())),
            preferred_element_type=jnp.float32,
        ) + b1_ref[...]
        h_ref[...] = jnp.maximum(h, 0.0)

    nc = 4
    rows = h_ref.shape[0] // nc
    for i in range(nc):
        sl = pl.ds(i * rows, rows)
        cw = jax.lax.dot_general(
            h_ref[sl, :], w2_ref[...],
            (((1,), (1,)), ((), ())),
            preferred_element_type=jnp.float32,
        ) + b2_ref[...]
        cw_ref[sl, :] = cw

        contrib = jnp.dot(cw, cb_ref[...], preferred_element_type=jnp.float32)

        @pl.when(k == 0)
        def _init_acc():
            vq_ref[sl, :] = contrib

        @pl.when(k > 0)
        def _acc():
            vq_ref[sl, :] += contrib


@functools.partial(jax.jit, static_argnames=("kb",))
def _fused(x, w1, b1, w2, b2, cb, kb=1024):
    n, c = x.shape
    mid = w1.shape[0]
    kk = w2.shape[0]
    grid = (kk // kb,)
    cw, vq = pl.pallas_call(
        _fused_body,
        grid=grid,
        in_specs=[
            pl.BlockSpec((n, c), lambda k: (0, 0)),
            pl.BlockSpec((mid, c), lambda k: (0, 0)),
            pl.BlockSpec((1, mid), lambda k: (0, 0)),
            pl.BlockSpec((kb, mid), lambda k: (k, 0)),
            pl.BlockSpec((1, kb), lambda k: (0, k)),
            pl.BlockSpec((kb, c), lambda k: (k, 0)),
        ],
        out_specs=[
            pl.BlockSpec((n, kb), lambda k: (0, k)),
            pl.BlockSpec((n, c), lambda k: (0, 0)),
        ],
        out_shape=[
            jax.ShapeDtypeStruct((n, kk), jnp.float32),
            jax.ShapeDtypeStruct((n, c), jnp.float32),
        ],
        scratch_shapes=[pltpu.VMEM((n, mid), jnp.float32)],
        compiler_params=pltpu.CompilerParams(
            dimension_semantics=("arbitrary",),
        ),
    )(x, w1, b1, w2, b2, cb)
    return cw, vq


def kernel(embed, W1, b1, W2, b2, codebook):
    Bx, Cx, Hx, Wx = embed.shape
    n = Bx * Hx * Wx
    x = jnp.transpose(embed, (0, 2, 3, 1)).reshape(n, Cx)
    cw, vq = _fused(x, W1, b1.reshape(1, -1), W2, b2.reshape(1, -1), codebook)
    code_weight = cw.reshape(Bx, Hx * Wx, -1)
    embed_vq = jnp.transpose(vq.reshape(Bx, Hx, Wx, Cx), (0, 3, 1, 2))
    return (embed_vq, code_weight, codebook)


# speedup vs baseline: 1.0624x; 1.0624x over previous
import functools

import jax
import jax.numpy as jnp
from jax.experimental import pallas as pl
from jax.experimental.pallas import tpu as pltpu


def _fused_body(x_ref, w1_ref, b1_ref, w2_ref, b2_ref, cb_ref,
                cw_ref, vq_ref, h_ref):
    k = pl.program_id(0)

    @pl.when(k == 0)
    def _compute_h():
        h = jax.lax.dot_general(
            x_ref[...], w1_ref[...],
            (((1,), (1,)), ((), ())),
            preferred_element_type=jnp.float32,
        ) + b1_ref[...]
        h_ref[...] = jnp.maximum(h, 0.0)

    nc = 2
    rows = h_ref.shape[0] // nc
    for i in range(nc):
        sl = pl.ds(i * rows, rows)
        cw = jax.lax.dot_general(
            h_ref[sl, :], w2_ref[...],
            (((1,), (1,)), ((), ())),
            preferred_element_type=jnp.float32,
        ) + b2_ref[...]
        cw_ref[sl, :] = cw

        contrib = jnp.dot(cw, cb_ref[...], preferred_element_type=jnp.float32)

        @pl.when(k == 0)
        def _init_acc():
            vq_ref[sl, :] = contrib

        @pl.when(k > 0)
        def _acc():
            vq_ref[sl, :] += contrib


@functools.partial(jax.jit, static_argnames=("kb",))
def _fused(x, w1, b1, w2, b2, cb, kb=1024):
    n, c = x.shape
    mid = w1.shape[0]
    kk = w2.shape[0]
    grid = (kk // kb,)
    cw, vq = pl.pallas_call(
        _fused_body,
        grid=grid,
        in_specs=[
            pl.BlockSpec((n, c), lambda k: (0, 0)),
            pl.BlockSpec((mid, c), lambda k: (0, 0)),
            pl.BlockSpec((1, mid), lambda k: (0, 0)),
            pl.BlockSpec((kb, mid), lambda k: (k, 0)),
            pl.BlockSpec((1, kb), lambda k: (0, k)),
            pl.BlockSpec((kb, c), lambda k: (k, 0)),
        ],
        out_specs=[
            pl.BlockSpec((n, kb), lambda k: (0, k)),
            pl.BlockSpec((n, c), lambda k: (0, 0)),
        ],
        out_shape=[
            jax.ShapeDtypeStruct((n, kk), jnp.float32),
            jax.ShapeDtypeStruct((n, c), jnp.float32),
        ],
        scratch_shapes=[pltpu.VMEM((n, mid), jnp.float32)],
        compiler_params=pltpu.CompilerParams(
            dimension_semantics=("arbitrary",),
        ),
    )(x, w1, b1, w2, b2, cb)
    return cw, vq


def kernel(embed, W1, b1, W2, b2, codebook):
    Bx, Cx, Hx, Wx = embed.shape
    n = Bx * Hx * Wx
    x = jnp.transpose(embed, (0, 2, 3, 1)).reshape(n, Cx)
    cw, vq = _fused(x, W1, b1.reshape(1, -1), W2, b2.reshape(1, -1), codebook)
    code_weight = cw.reshape(Bx, Hx * Wx, -1)
    embed_vq = jnp.transpose(vq.reshape(Bx, Hx, Wx, Cx), (0, 3, 1, 2))
    return (embed_vq, code_weight, codebook)


# speedup vs baseline: 1.0974x; 1.0330x over previous
import functools

import jax
import jax.numpy as jnp
from jax.experimental import pallas as pl
from jax.experimental.pallas import tpu as pltpu


def _fused_body(x_ref, w1_ref, b1_ref, w2_ref, b2_ref, cb_ref,
                cw_ref, vq_ref, h_ref):
    k = pl.program_id(0)

    @pl.when(k == 0)
    def _compute_h():
        h = jax.lax.dot_general(
            x_ref[...], w1_ref[...],
            (((1,), (1,)), ((), ())),
            preferred_element_type=jnp.float32,
        ) + b1_ref[...]
        h_ref[...] = jnp.maximum(h, 0.0)

    cw = jax.lax.dot_general(
        h_ref[...], w2_ref[...],
        (((1,), (1,)), ((), ())),
        preferred_element_type=jnp.float32,
    ) + b2_ref[...]
    cw_ref[...] = cw

    contrib = jnp.dot(cw, cb_ref[...], preferred_element_type=jnp.float32)

    @pl.when(k == 0)
    def _init_acc():
        vq_ref[...] = contrib

    @pl.when(k > 0)
    def _acc():
        vq_ref[...] += contrib


@functools.partial(jax.jit, static_argnames=("kb",))
def _fused(x, w1, b1, w2, b2, cb, kb=1024):
    n, c = x.shape
    mid = w1.shape[0]
    kk = w2.shape[0]
    grid = (kk // kb,)
    cw, vq = pl.pallas_call(
        _fused_body,
        grid=grid,
        in_specs=[
            pl.BlockSpec((n, c), lambda k: (0, 0)),
            pl.BlockSpec((mid, c), lambda k: (0, 0)),
            pl.BlockSpec((1, mid), lambda k: (0, 0)),
            pl.BlockSpec((kb, mid), lambda k: (k, 0)),
            pl.BlockSpec((1, kb), lambda k: (0, k)),
            pl.BlockSpec((kb, c), lambda k: (k, 0)),
        ],
        out_specs=[
            pl.BlockSpec((n, kb), lambda k: (0, k)),
            pl.BlockSpec((n, c), lambda k: (0, 0)),
        ],
        out_shape=[
            jax.ShapeDtypeStruct((n, kk), jnp.float32),
            jax.ShapeDtypeStruct((n, c), jnp.float32),
        ],
        scratch_shapes=[pltpu.VMEM((n, mid), jnp.float32)],
        compiler_params=pltpu.CompilerParams(
            dimension_semantics=("arbitrary",),
        ),
    )(x, w1, b1, w2, b2, cb)
    return cw, vq


def kernel(embed, W1, b1, W2, b2, codebook):
    Bx, Cx, Hx, Wx = embed.shape
    n = Bx * Hx * Wx
    x = jnp.transpose(embed, (0, 2, 3, 1)).reshape(n, Cx)
    cw, vq = _fused(x, W1, b1.reshape(1, -1), W2, b2.reshape(1, -1), codebook)
    code_weight = cw.reshape(Bx, Hx * Wx, -1)
    embed_vq = jnp.transpose(vq.reshape(Bx, Hx, Wx, Cx), (0, 3, 1, 2))
    return (embed_vq, code_weight, codebook)
